# Initial kernel scaffold; baseline (speedup 1.0000x reference)
import jax
import jax.numpy as jnp
from jax.experimental import pallas as pl


def kernel(host_node_ids, nf_x, e_h2f_src, e_h2f_dst, e_f2h_src, e_f2h_dst, e_f2f, nf_batch, emb_table, l0_h2f_Wrel, l0_h2f_Wroot, l0_h2f_b, l0_f2h_Wrel, l0_f2h_Wroot, l0_f2h_b, l0_f2f_Wrel, l0_f2f_Wroot, l0_f2f_b, l1_h2f_Wrel, l1_h2f_Wroot, l1_h2f_b, l1_f2h_Wrel, l1_f2h_Wroot, l1_f2h_b, l1_f2f_Wrel, l1_f2f_Wroot, l1_f2f_b, cls_W1, cls_b1, cls_W2, cls_b2, cls_W3, cls_b3):
    raise NotImplementedError("write your pallas kernel here")


# speedup vs baseline: 1.0000x; 1.0000x over previous
import jax
import jax.numpy as jnp
from jax.experimental import pallas as pl
from jax.experimental.pallas import tpu as pltpu

H = 128
F = 97
NH = 4000
NF = 50000
E = 500000
NC = 10
NB = 64


def _gconv(x_src, x_dst, src, dst, Wrel, Wroot, b, n_dst):
    agg = jax.ops.segment_sum(x_src[src], dst, num_segments=n_dst)
    return agg @ Wrel + b + x_dst @ Wroot


def _mlp_kernel(pooled_ref, w1_ref, b1_ref, w2_ref, b2_ref, w3_ref, b3_ref, out_ref):
    h1 = jax.nn.relu(pooled_ref[...] @ w1_ref[...] + b1_ref[...])
    h2 = jax.nn.relu(h1 @ w2_ref[...] + b2_ref[...])
    out_ref[...] = h2 @ w3_ref[...] + b3_ref[...]


def kernel(host_node_ids, nf_x, e_h2f_src, e_h2f_dst, e_f2h_src, e_f2h_dst, e_f2f, nf_batch, emb_table, l0_h2f_Wrel, l0_h2f_Wroot, l0_h2f_b, l0_f2h_Wrel, l0_f2h_Wroot, l0_f2h_b, l0_f2f_Wrel, l0_f2f_Wroot, l0_f2f_b, l1_h2f_Wrel, l1_h2f_Wroot, l1_h2f_b, l1_f2h_Wrel, l1_f2h_Wroot, l1_f2h_b, l1_f2f_Wrel, l1_f2f_Wroot, l1_f2f_b, cls_W1, cls_b1, cls_W2, cls_b2, cls_W3, cls_b3):
    x_h = emb_table[host_node_ids]
    x_f = nf_x
    o_f = _gconv(x_h, x_f, e_h2f_src, e_h2f_dst, l0_h2f_Wrel, l0_h2f_Wroot, l0_h2f_b, NF)
    o_f = o_f + _gconv(x_f, x_f, e_f2f[0], e_f2f[1], l0_f2f_Wrel, l0_f2f_Wroot, l0_f2f_b, NF)
    o_h = _gconv(x_f, x_h, e_f2h_src, e_f2h_dst, l0_f2h_Wrel, l0_f2h_Wroot, l0_f2h_b, NH)
    x_f = jax.nn.relu(o_f)
    x_h = jax.nn.relu(o_h)
    o_f = _gconv(x_h, x_f, e_h2f_src, e_h2f_dst, l1_h2f_Wrel, l1_h2f_Wroot, l1_h2f_b, NF)
    o_f = o_f + _gconv(x_f, x_f, e_f2f[0], e_f2f[1], l1_f2f_Wrel, l1_f2f_Wroot, l1_f2f_b, NF)
    x_f = o_f
    pooled = jax.ops.segment_max(x_f, nf_batch, num_segments=NB)

    return pl.pallas_call(
        _mlp_kernel,
        out_shape=jax.ShapeDtypeStruct((NB, NC), jnp.float32),
    )(pooled, cls_W1, cls_b1, cls_W2, cls_b2, cls_W3, cls_b3)


# speedup vs baseline: 2.0558x; 2.0557x over previous
import functools

import jax
import jax.numpy as jnp
from jax import lax
from jax.experimental import pallas as pl
from jax.experimental.pallas import tpu as pltpu
from jax.experimental.pallas import tpu_sc as plsc

H = 128
F = 97
NH = 4000
NF = 50000
E = 500000
NC = 10
NB = 64

_NSC = 2
_NSUB = 16
_NW = _NSC * _NSUB

_W = 64
_HWIN_PW = -(-E // (_W * _NW))
_HWIN_TOT = _HWIN_PW * _NW
_EPAD = _HWIN_TOT * _W
_NHP = 4096


def _pad_edges(src, dst, n_dst_real, n_src_real, n_pad_total):
    npad = n_pad_total - src.shape[0]
    pad_pos = jnp.arange(npad, dtype=jnp.int32)
    pad_src = pad_pos * 37 % n_src_real
    pad_dst = n_dst_real + (pad_pos % 16)
    return (jnp.concatenate([src.astype(jnp.int32), pad_src]),
            jnp.concatenate([dst.astype(jnp.int32), pad_dst]))


def _f2h_agg_kernel(src2d, dst2d, init, msrc, out, src_st, dst_st, rows, gsem, ssem, acc):
    c = lax.axis_index("c")
    s = lax.axis_index("s")
    wid = c * _NSUB + s
    rows_per_tile = _NHP // _NSUB
    r0 = s * rows_per_tile

    pltpu.sync_copy(src2d.at[wid], src_st)
    pltpu.sync_copy(dst2d.at[wid], dst_st)
    pltpu.sync_copy(init.at[c, pl.ds(r0, rows_per_tile)], acc.at[pl.ds(r0, rows_per_tile)])
    plsc.subcore_barrier()

    pltpu.async_copy(msrc.at[src_st.at[0]], rows.at[0], gsem.at[0])
    pltpu.async_copy(msrc.at[src_st.at[1]], rows.at[1], gsem.at[1])

    def body(k, _):
        b = k % 2
        pltpu.make_async_copy(msrc.at[src_st.at[k]], rows.at[b], gsem.at[b]).wait()
        pltpu.async_copy(rows.at[b], acc.at[dst_st.at[k]], ssem.at[b], add=True).wait()

        @pl.when(k + 2 < _HWIN_PW)
        def _():
            pltpu.async_copy(msrc.at[src_st.at[k + 2]], rows.at[b], gsem.at[b])
        return ()

    lax.fori_loop(0, _HWIN_PW, body, (), unroll=False)
    plsc.subcore_barrier()
    pltpu.sync_copy(acc.at[pl.ds(r0, rows_per_tile)], out.at[c, pl.ds(r0, rows_per_tile)])


_FS = 12544
_FDUM = 128
_FACC = _FS + _FDUM
_NFP = 4 * _FS
_FWPT = 1024
_FBLK = 16
_FEPAD = _FWPT * _NSUB * _W


def _f_agg_kernel(src3d, dst3d, init, gtab, out, src_st, dst_st, dstw, rows,
                  gsem, ssem, acc):
    c = lax.axis_index("c")
    s = lax.axis_index("s")
    rpt = _FACC // _NSUB
    r0 = s * rpt

    def chunk_pass(cc, _):
        chunk = 2 * c + cc
        base = chunk * _FS
        pltpu.sync_copy(init.at[chunk, pl.ds(r0, rpt)], acc.at[pl.ds(r0, rpt)])
        plsc.subcore_barrier()

        def stage(blk):
            pltpu.sync_copy(src3d.at[s, pl.ds(blk * _FBLK, _FBLK)], src_st.at[blk % 2])
            pltpu.sync_copy(dst3d.at[s, pl.ds(blk * _FBLK, _FBLK)], dst_st.at[blk % 2])

        stage(0)
        pltpu.async_copy(gtab.at[src_st.at[0, 0]], rows.at[0], gsem.at[0])
        pltpu.async_copy(gtab.at[src_st.at[0, 1]], rows.at[1], gsem.at[1])

        def body(w, _):
            b = w % 2
            p = (w // _FBLK) % 2
            wm = w % _FBLK
            for j in range(_W // 16):
                v = dst_st[p, wm, pl.ds(j * 16, 16)]
                m = (v >= base) & (v < base + _FS)
                rel = jnp.where(m, v - base, _FS + j * 16 + lax.iota(jnp.int32, 16))
                dstw[b, pl.ds(j * 16, 16)] = rel

            pltpu.make_async_copy(gtab.at[src_st.at[p, wm]], rows.at[b], gsem.at[b]).wait()
            pltpu.async_copy(rows.at[b], acc.at[dstw.at[b]], ssem.at[b], add=True).wait()

            @pl.when(w + 2 < _FWPT)
            def _():
                nw = w + 2

                @pl.when(nw % _FBLK == 0)
                def _():
                    stage(nw // _FBLK)

                pltpu.async_copy(gtab.at[src_st.at[(nw // _FBLK) % 2, nw % _FBLK]],
                                 rows.at[b], gsem.at[b])
            return ()

        lax.fori_loop(0, _FWPT, body, (), unroll=False)
        plsc.subcore_barrier()
        drpt = _FS // _NSUB
        pltpu.sync_copy(acc.at[pl.ds(s * drpt, drpt)], out.at[chunk, pl.ds(s * drpt, drpt)])
        plsc.subcore_barrier()
        return ()

    lax.fori_loop(0, 2, chunk_pass, (), unroll=False)


def _f_agg(gtab, src_g, dst_g, init_full):
    src3d = src_g.reshape(_NSUB, _FWPT, _W)
    dst3d = dst_g.reshape(_NSUB, _FWPT, _W)
    initp = jnp.zeros((4, _FACC, H), jnp.float32).at[:, :_FS].set(
        jnp.pad(init_full, ((0, _NFP - NF), (0, 0))).reshape(4, _FS, H))
    k = pl.kernel(
        _f_agg_kernel,
        out_type=jax.ShapeDtypeStruct((4, _FS, H), jnp.float32),
        mesh=plsc.VectorSubcoreMesh(core_axis_name="c", subcore_axis_name="s"),
        scratch_types=[
            pltpu.VMEM((2, _FBLK, _W), jnp.int32),
            pltpu.VMEM((2, _FBLK, _W), jnp.int32),
            pltpu.VMEM((2, _W), jnp.int32),
            pltpu.VMEM((2, _W, H), jnp.float32),
            pltpu.SemaphoreType.DMA((2,)),
            pltpu.SemaphoreType.DMA((2,)),
            pltpu.VMEM_SHARED((_FACC, H), jnp.float32),
        ],
    )
    out = k(src3d, dst3d, initp, gtab)
    return out.reshape(_NFP, H)[:NF]


def _f2h_agg(msrc, e_src, e_dst, init_row):
    src_p, dst_p = _pad_edges(e_src, e_dst, NH, NF, _EPAD)
    src2d = src_p.reshape(_NW, _HWIN_PW, _W)
    dst2d = dst_p.reshape(_NW, _HWIN_PW, _W)
    init = jnp.stack([
        jnp.pad(init_row, ((0, _NHP - NH), (0, 0))),
        jnp.zeros((_NHP, H), jnp.float32),
    ])
    k = pl.kernel(
        _f2h_agg_kernel,
        out_type=jax.ShapeDtypeStruct((2, _NHP, H), jnp.float32),
        mesh=plsc.VectorSubcoreMesh(core_axis_name="c", subcore_axis_name="s"),
        scratch_types=[
            pltpu.VMEM((_HWIN_PW, _W), jnp.int32),
            pltpu.VMEM((_HWIN_PW, _W), jnp.int32),
            pltpu.VMEM((2, _W, H), jnp.float32),
            pltpu.SemaphoreType.DMA((2,)),
            pltpu.SemaphoreType.DMA((2,)),
            pltpu.VMEM_SHARED((_NHP, H), jnp.float32),
        ],
    )
    out = k(src2d, dst2d, init, msrc)
    return out[0, :NH] + out[1, :NH]


def _gconv(x_src, x_dst, src, dst, Wrel, Wroot, b, n_dst):
    agg = jax.ops.segment_sum(x_src[src], dst, num_segments=n_dst)
    return agg @ Wrel + b + x_dst @ Wroot


def _mlp_kernel(pooled_ref, w1_ref, b1_ref, w2_ref, b2_ref, w3_ref, b3_ref, out_ref):
    h1 = jax.nn.relu(pooled_ref[...] @ w1_ref[...] + b1_ref[...])
    h2 = jax.nn.relu(h1 @ w2_ref[...] + b2_ref[...])
    out_ref[...] = h2 @ w3_ref[...] + b3_ref[...]


def kernel(host_node_ids, nf_x, e_h2f_src, e_h2f_dst, e_f2h_src, e_f2h_dst, e_f2f, nf_batch, emb_table, l0_h2f_Wrel, l0_h2f_Wroot, l0_h2f_b, l0_f2h_Wrel, l0_f2h_Wroot, l0_f2h_b, l0_f2f_Wrel, l0_f2f_Wroot, l0_f2f_b, l1_h2f_Wrel, l1_h2f_Wroot, l1_h2f_b, l1_f2h_Wrel, l1_f2h_Wroot, l1_f2h_b, l1_f2f_Wrel, l1_f2f_Wroot, l1_f2f_b, cls_W1, cls_b1, cls_W2, cls_b2, cls_W3, cls_b3):
    x_h = emb_table[host_node_ids]
    x_f = nf_x

    src_g, dst_g = _pad_edges(
        jnp.concatenate([e_h2f_src.astype(jnp.int32),
                         e_f2f[0].astype(jnp.int32) + NH]),
        jnp.concatenate([e_h2f_dst.astype(jnp.int32),
                         e_f2f[1].astype(jnp.int32)]),
        NF, NH + NF, _FEPAD)

    gtab0 = jnp.concatenate([x_h @ l0_h2f_Wrel, x_f @ l0_f2f_Wrel])
    z0 = x_f @ (l0_h2f_Wroot + l0_f2f_Wroot) + (l0_h2f_b + l0_f2f_b)
    o_f = _f_agg(gtab0, src_g, dst_g, z0)
    o_h = _f2h_agg(x_f @ l0_f2h_Wrel, e_f2h_src, e_f2h_dst,
                   x_h @ l0_f2h_Wroot + l0_f2h_b)
    x_f = jax.nn.relu(o_f)
    x_h = jax.nn.relu(o_h)

    gtab1 = jnp.concatenate([x_h @ l1_h2f_Wrel, x_f @ l1_f2f_Wrel])
    z1 = x_f @ (l1_h2f_Wroot + l1_f2f_Wroot) + (l1_h2f_b + l1_f2f_b)
    x_f = _f_agg(gtab1, src_g, dst_g, z1)
    pooled = jax.ops.segment_max(x_f, nf_batch, num_segments=NB)

    return pl.pallas_call(
        _mlp_kernel,
        out_shape=jax.ShapeDtypeStruct((NB, NC), jnp.float32),
    )(pooled, cls_W1, cls_b1, cls_W2, cls_b2, cls_W3, cls_b3)


# speedup vs baseline: 2.0971x; 1.0201x over previous
import functools

import jax
import jax.numpy as jnp
from jax import lax
from jax.experimental import pallas as pl
from jax.experimental.pallas import tpu as pltpu
from jax.experimental.pallas import tpu_sc as plsc

H = 128
F = 97
NH = 4000
NF = 50000
E = 500000
NC = 10
NB = 64

_NSC = 2
_NSUB = 16
_NW = _NSC * _NSUB

_W = 64
_HWIN_PW = -(-E // (_W * _NW))
_HWIN_TOT = _HWIN_PW * _NW
_EPAD = _HWIN_TOT * _W
_NHP = 4096


def _pad_edges(src, dst, n_dst_real, n_src_real, n_pad_total):
    npad = n_pad_total - src.shape[0]
    pad_pos = jnp.arange(npad, dtype=jnp.int32)
    pad_src = pad_pos * 37 % n_src_real
    pad_dst = n_dst_real + (pad_pos % 16)
    return (jnp.concatenate([src.astype(jnp.int32), pad_src]),
            jnp.concatenate([dst.astype(jnp.int32), pad_dst]))


def _f2h_agg_kernel(src2d, dst2d, init, msrc, out, src_st, dst_st, rows, gsem, ssem, acc):
    c = lax.axis_index("c")
    s = lax.axis_index("s")
    wid = c * _NSUB + s
    rows_per_tile = _NHP // _NSUB
    r0 = s * rows_per_tile

    pltpu.sync_copy(src2d.at[wid], src_st)
    pltpu.sync_copy(dst2d.at[wid], dst_st)
    pltpu.sync_copy(init.at[c, pl.ds(r0, rows_per_tile)], acc.at[pl.ds(r0, rows_per_tile)])
    plsc.subcore_barrier()

    pltpu.async_copy(msrc.at[src_st.at[0]], rows.at[0], gsem.at[0])
    pltpu.async_copy(msrc.at[src_st.at[1]], rows.at[1], gsem.at[1])

    def body(k, _):
        b = k % 2
        pltpu.make_async_copy(msrc.at[src_st.at[k]], rows.at[b], gsem.at[b]).wait()
        pltpu.async_copy(rows.at[b], acc.at[dst_st.at[k]], ssem.at[b], add=True).wait()

        @pl.when(k + 2 < _HWIN_PW)
        def _():
            pltpu.async_copy(msrc.at[src_st.at[k + 2]], rows.at[b], gsem.at[b])
        return ()

    lax.fori_loop(0, _HWIN_PW, body, (), unroll=False)
    plsc.subcore_barrier()
    pltpu.sync_copy(acc.at[pl.ds(r0, rows_per_tile)], out.at[c, pl.ds(r0, rows_per_tile)])


_FS = 12544
_FDUM = 128
_FACC = _FS + _FDUM
_NFP = 4 * _FS
_FWPT = 1024
_FBLK = 16
_FEPAD = _FWPT * _NSUB * _W


def _f_agg_kernel(src3d, dst3d, init, gtab, out, src_st, dst_st, dstw, rows,
                  gsem, ssem, acc):
    c = lax.axis_index("c")
    s = lax.axis_index("s")
    rpt = _FACC // _NSUB
    r0 = s * rpt

    def chunk_pass(cc, _):
        chunk = 2 * c + cc
        base = chunk * _FS
        pltpu.sync_copy(init.at[chunk, pl.ds(r0, rpt)], acc.at[pl.ds(r0, rpt)])
        plsc.subcore_barrier()

        def stage(blk):
            pltpu.sync_copy(src3d.at[s, pl.ds(blk * _FBLK, _FBLK)], src_st.at[blk % 2])
            pltpu.sync_copy(dst3d.at[s, pl.ds(blk * _FBLK, _FBLK)], dst_st.at[blk % 2])

        stage(0)
        pltpu.async_copy(gtab.at[src_st.at[0, 0]], rows.at[0], gsem.at[0])
        pltpu.async_copy(gtab.at[src_st.at[0, 1]], rows.at[1], gsem.at[1])

        def body(w, _):
            b = w % 2
            p = (w // _FBLK) % 2
            wm = w % _FBLK
            for j in range(_W // 16):
                v = dst_st[p, wm, pl.ds(j * 16, 16)]
                m = (v >= base) & (v < base + _FS)
                rel = jnp.where(m, v - base, _FS + j * 16 + lax.iota(jnp.int32, 16))
                dstw[b, pl.ds(j * 16, 16)] = rel

            pltpu.make_async_copy(gtab.at[src_st.at[p, wm]], rows.at[b], gsem.at[b]).wait()
            pltpu.async_copy(rows.at[b], acc.at[dstw.at[b]], ssem.at[b], add=True).wait()

            @pl.when(w + 2 < _FWPT)
            def _():
                nw = w + 2

                @pl.when(nw % _FBLK == 0)
                def _():
                    stage(nw // _FBLK)

                pltpu.async_copy(gtab.at[src_st.at[(nw // _FBLK) % 2, nw % _FBLK]],
                                 rows.at[b], gsem.at[b])
            return ()

        lax.fori_loop(0, _FWPT, body, (), unroll=False)
        plsc.subcore_barrier()
        drpt = _FS // _NSUB
        pltpu.sync_copy(acc.at[pl.ds(s * drpt, drpt)], out.at[chunk, pl.ds(s * drpt, drpt)])
        plsc.subcore_barrier()
        return ()

    lax.fori_loop(0, 2, chunk_pass, (), unroll=False)


def _f_agg(gtab, src_g, dst_g, init_full):
    src3d = src_g.reshape(_NSUB, _FWPT, _W)
    dst3d = dst_g.reshape(_NSUB, _FWPT, _W)
    initp = jnp.zeros((4, _FACC, H), jnp.float32).at[:, :_FS].set(
        jnp.pad(init_full, ((0, _NFP - NF), (0, 0))).reshape(4, _FS, H))
    k = pl.kernel(
        _f_agg_kernel,
        out_type=jax.ShapeDtypeStruct((4, _FS, H), jnp.float32),
        mesh=plsc.VectorSubcoreMesh(core_axis_name="c", subcore_axis_name="s"),
        scratch_types=[
            pltpu.VMEM((2, _FBLK, _W), jnp.int32),
            pltpu.VMEM((2, _FBLK, _W), jnp.int32),
            pltpu.VMEM((2, _W), jnp.int32),
            pltpu.VMEM((2, _W, H), jnp.float32),
            pltpu.SemaphoreType.DMA((2,)),
            pltpu.SemaphoreType.DMA((2,)),
            pltpu.VMEM_SHARED((_FACC, H), jnp.float32),
        ],
    )
    out = k(src3d, dst3d, initp, gtab)
    return out.reshape(_NFP, H)[:NF]


def _f2h_agg(msrc, e_src, e_dst, init_row):
    src_p, dst_p = _pad_edges(e_src, e_dst, NH, NF, _EPAD)
    src2d = src_p.reshape(_NW, _HWIN_PW, _W)
    dst2d = dst_p.reshape(_NW, _HWIN_PW, _W)
    init = jnp.stack([
        jnp.pad(init_row, ((0, _NHP - NH), (0, 0))),
        jnp.zeros((_NHP, H), jnp.float32),
    ])
    k = pl.kernel(
        _f2h_agg_kernel,
        out_type=jax.ShapeDtypeStruct((2, _NHP, H), jnp.float32),
        mesh=plsc.VectorSubcoreMesh(core_axis_name="c", subcore_axis_name="s"),
        scratch_types=[
            pltpu.VMEM((_HWIN_PW, _W), jnp.int32),
            pltpu.VMEM((_HWIN_PW, _W), jnp.int32),
            pltpu.VMEM((2, _W, H), jnp.float32),
            pltpu.SemaphoreType.DMA((2,)),
            pltpu.SemaphoreType.DMA((2,)),
            pltpu.VMEM_SHARED((_NHP, H), jnp.float32),
        ],
    )
    out = k(src2d, dst2d, init, msrc)
    return out[:, :NH]


def _gconv(x_src, x_dst, src, dst, Wrel, Wroot, b, n_dst):
    agg = jax.ops.segment_sum(x_src[src], dst, num_segments=n_dst)
    return agg @ Wrel + b + x_dst @ Wroot


def _mm_kernel(x_ref, w_ref, b_ref, o_ref, *, relu_x):
    x = x_ref[...]
    if relu_x:
        x = jnp.maximum(x, 0.0)
    o_ref[...] = jnp.dot(x, w_ref[...], preferred_element_type=jnp.float32) + b_ref[...]


def _mm(x, w, b, relu_x=False, bm=2000):
    m, kdim = x.shape
    n = w.shape[1]
    return pl.pallas_call(
        functools.partial(_mm_kernel, relu_x=relu_x),
        grid=(m // bm,),
        in_specs=[pl.BlockSpec((bm, kdim), lambda i: (i, 0)),
                  pl.BlockSpec((kdim, n), lambda i: (0, 0)),
                  pl.BlockSpec((1, n), lambda i: (0, 0))],
        out_specs=pl.BlockSpec((bm, n), lambda i: (i, 0)),
        out_shape=jax.ShapeDtypeStruct((m, n), jnp.float32),
    )(x, w, b.reshape(1, n))


def _h1_kernel(oh_ref, w_ref, o_ref):
    xh = jnp.maximum(oh_ref[0] + oh_ref[1], 0.0)
    o_ref[...] = jnp.dot(xh, w_ref[...], preferred_element_type=jnp.float32)


def _h1_mm(oh2, w):
    return pl.pallas_call(
        _h1_kernel,
        out_shape=jax.ShapeDtypeStruct((NH, H), jnp.float32),
    )(oh2, w)


_NFSEG = 51200


def _final_kernel(x_ref, bid_ref, w1_ref, b1_ref, w2_ref, b2_ref, w3_ref, b3_ref,
                  out_ref, acc_ref):
    i = pl.program_id(0)
    nblk = pl.num_programs(0)

    @pl.when(i == 0)
    def _():
        acc_ref[...] = jnp.full((NB, H), -jnp.inf, jnp.float32)

    x = x_ref[...]
    bid = bid_ref[...]
    lo = jnp.min(bid)
    hi = jnp.max(bid)
    for s in range(NB):
        @pl.when(jnp.logical_and(lo <= s, s <= hi))
        def _():
            m = jnp.max(jnp.where(bid == s, x, -jnp.inf), axis=0)
            acc_ref[s, :] = jnp.maximum(acc_ref[s, :], m)

    @pl.when(i == nblk - 1)
    def _():
        h1 = jnp.maximum(jnp.dot(acc_ref[...], w1_ref[...],
                                 preferred_element_type=jnp.float32) + b1_ref[...], 0.0)
        h2 = jnp.maximum(jnp.dot(h1, w2_ref[...],
                                 preferred_element_type=jnp.float32) + b2_ref[...], 0.0)
        out_ref[...] = jnp.dot(h2, w3_ref[...],
                               preferred_element_type=jnp.float32) + b3_ref[...]


def _segmax_mlp(x_f, nf_batch, w1, b1, w2, b2, w3, b3):
    xp = jnp.pad(x_f, ((0, _NFSEG - NF), (0, 0)))
    bidb = jnp.broadcast_to(
        jnp.pad(nf_batch.astype(jnp.float32), (0, _NFSEG - NF),
                constant_values=float(NB))[:, None], (_NFSEG, H))
    bm = 2048
    return pl.pallas_call(
        _final_kernel,
        grid=(_NFSEG // bm,),
        in_specs=[pl.BlockSpec((bm, H), lambda i: (i, 0)),
                  pl.BlockSpec((bm, H), lambda i: (i, 0)),
                  pl.BlockSpec((H, H // 2), lambda i: (0, 0)),
                  pl.BlockSpec((1, H // 2), lambda i: (0, 0)),
                  pl.BlockSpec((H // 2, H), lambda i: (0, 0)),
                  pl.BlockSpec((1, H), lambda i: (0, 0)),
                  pl.BlockSpec((H, NC), lambda i: (0, 0)),
                  pl.BlockSpec((1, NC), lambda i: (0, 0))],
        out_specs=pl.BlockSpec((NB, NC), lambda i: (0, 0)),
        out_shape=jax.ShapeDtypeStruct((NB, NC), jnp.float32),
        scratch_shapes=[pltpu.VMEM((NB, H), jnp.float32)],
    )(xp, bidb, w1, b1.reshape(1, -1), w2, b2.reshape(1, -1), w3, b3.reshape(1, -1))


def _hgather_kernel(ew, ids2d, out, idx_v, rows_v, sem):
    c = lax.axis_index("c")
    s = lax.axis_index("s")
    wid = c * _NSUB + s
    pltpu.sync_copy(ids2d.at[wid], idx_v)
    pltpu.async_copy(ew.at[idx_v], rows_v, sem).wait()
    pltpu.sync_copy(rows_v, out.at[wid])


def _hgather(ew, ids):
    idp = jnp.pad(ids.astype(jnp.int32), (0, 4096 - NH)).reshape(_NW, 128)
    k = pl.kernel(
        _hgather_kernel,
        out_type=jax.ShapeDtypeStruct((_NW, 128, 2 * H), jnp.float32),
        mesh=plsc.VectorSubcoreMesh(core_axis_name="c", subcore_axis_name="s"),
        scratch_types=[
            pltpu.VMEM((128,), jnp.int32),
            pltpu.VMEM((128, 2 * H), jnp.float32),
            pltpu.SemaphoreType.DMA,
        ],
    )
    return k(ew, idp).reshape(4096, 2 * H)


def kernel(host_node_ids, nf_x, e_h2f_src, e_h2f_dst, e_f2h_src, e_f2h_dst, e_f2f, nf_batch, emb_table, l0_h2f_Wrel, l0_h2f_Wroot, l0_h2f_b, l0_f2h_Wrel, l0_f2h_Wroot, l0_f2h_b, l0_f2f_Wrel, l0_f2f_Wroot, l0_f2f_b, l1_h2f_Wrel, l1_h2f_Wroot, l1_h2f_b, l1_f2h_Wrel, l1_f2h_Wroot, l1_f2h_b, l1_f2f_Wrel, l1_f2f_Wroot, l1_f2f_b, cls_W1, cls_b1, cls_W2, cls_b2, cls_W3, cls_b3):
    src_g, dst_g = _pad_edges(
        jnp.concatenate([e_h2f_src.astype(jnp.int32),
                         e_f2f[0].astype(jnp.int32) + NH]),
        jnp.concatenate([e_h2f_dst.astype(jnp.int32),
                         e_f2f[1].astype(jnp.int32)]),
        NF, NH + NF, _FEPAD)

    ew = _mm(emb_table, jnp.concatenate([l0_h2f_Wrel, l0_f2h_Wroot], axis=1),
             jnp.concatenate([jnp.zeros_like(l0_h2f_b), l0_f2h_b]), bm=4000)
    g = _hgather(ew, host_node_ids)
    f0 = _mm(nf_x,
             jnp.concatenate([l0_f2f_Wrel, l0_f2h_Wrel,
                              l0_h2f_Wroot + l0_f2f_Wroot], axis=1),
             jnp.concatenate([jnp.zeros((2 * H,), jnp.float32),
                              l0_h2f_b + l0_f2f_b]))

    gtab0 = jnp.concatenate([g[:NH, :H], f0[:, :H]])
    o_f0 = _f_agg(gtab0, src_g, dst_g, f0[:, 2 * H:])
    oh2 = _f2h_agg(f0[:, H:2 * H], e_f2h_src, e_f2h_dst, g[:NH, H:])

    t1 = _h1_mm(oh2, l1_h2f_Wrel)
    f1 = _mm(o_f0,
             jnp.concatenate([l1_f2f_Wrel, l1_h2f_Wroot + l1_f2f_Wroot], axis=1),
             jnp.concatenate([jnp.zeros((H,), jnp.float32),
                              l1_h2f_b + l1_f2f_b]), relu_x=True)
    gtab1 = jnp.concatenate([t1, f1[:, :H]])
    o_f1 = _f_agg(gtab1, src_g, dst_g, f1[:, H:])

    return _segmax_mlp(o_f1, nf_batch, cls_W1, cls_b1, cls_W2, cls_b2,
                       cls_W3, cls_b3)


# speedup vs baseline: 2.6116x; 1.2454x over previous
import functools

import jax
import jax.numpy as jnp
from jax import lax
from jax.experimental import pallas as pl
from jax.experimental.pallas import tpu as pltpu
from jax.experimental.pallas import tpu_sc as plsc

H = 128
F = 97
NH = 4000
NF = 50000
E = 500000
NC = 10
NB = 64

_NSC = 2
_NSUB = 16
_NW = _NSC * _NSUB

_W = 64
_HWIN_PW = -(-E // (_W * _NW))
_HWIN_TOT = _HWIN_PW * _NW
_EPAD = _HWIN_TOT * _W
_NHP = 4096


def _pad_edges(src, dst, n_dst_real, n_src_real, n_pad_total):
    npad = n_pad_total - src.shape[0]
    pad_pos = jnp.arange(npad, dtype=jnp.int32)
    pad_src = pad_pos * 37 % n_src_real
    pad_dst = n_dst_real + (pad_pos % 16)
    return (jnp.concatenate([src.astype(jnp.int32), pad_src]),
            jnp.concatenate([dst.astype(jnp.int32), pad_dst]))


def _f2h_agg_kernel(src2d, dst2d, init, msrc, out, src_st, dst_st, rows, gsem, ssem, acc):
    c = lax.axis_index("c")
    s = lax.axis_index("s")
    wid = c * _NSUB + s
    rows_per_tile = _NHP // _NSUB
    r0 = s * rows_per_tile

    pltpu.sync_copy(src2d.at[wid], src_st)
    pltpu.sync_copy(dst2d.at[wid], dst_st)
    pltpu.sync_copy(init.at[c, pl.ds(r0, rows_per_tile)], acc.at[pl.ds(r0, rows_per_tile)])
    plsc.subcore_barrier()

    pltpu.async_copy(msrc.at[src_st.at[0]], rows.at[0], gsem.at[0])
    pltpu.async_copy(msrc.at[src_st.at[1]], rows.at[1], gsem.at[1])

    def body(k, _):
        b = k % 2
        pltpu.make_async_copy(msrc.at[src_st.at[k]], rows.at[b], gsem.at[b]).wait()
        pltpu.async_copy(rows.at[b], acc.at[dst_st.at[k]], ssem.at[b], add=True).wait()

        @pl.when(k + 2 < _HWIN_PW)
        def _():
            pltpu.async_copy(msrc.at[src_st.at[k + 2]], rows.at[b], gsem.at[b])
        return ()

    lax.fori_loop(0, _HWIN_PW, body, (), unroll=False)
    plsc.subcore_barrier()
    pltpu.sync_copy(acc.at[pl.ds(r0, rows_per_tile)], out.at[c, pl.ds(r0, rows_per_tile)])


_FS = 25088
_FACC = _FS + 128
_NFP = 2 * _FS
_FW = 128
_FWPT = 512
_FBLK = 4
_FEPAD = _FWPT * _NSUB * _FW
_HF = H // 2


def _f_agg_kernel(src3d, dst3d, init, gtab2, out, src_st, dst_st, srcw, dstw,
                  rows, gsem, ssem, acc):
    c = lax.axis_index("c")
    s = lax.axis_index("s")
    rpt = _FACC // _NSUB
    r0 = s * rpt
    srcoff = c * (NH + NF)

    def half_pass(d, _):
        base = d * _FS
        pltpu.sync_copy(init.at[c, d, pl.ds(r0, rpt)], acc.at[pl.ds(r0, rpt)])
        plsc.subcore_barrier()

        def stage(blk):
            pltpu.sync_copy(src3d.at[s, pl.ds(blk * _FBLK, _FBLK)], src_st.at[blk % 2])
            pltpu.sync_copy(dst3d.at[s, pl.ds(blk * _FBLK, _FBLK)], dst_st.at[blk % 2])

        def fix(w, b):
            p = (w // _FBLK) % 2
            wm = w % _FBLK
            for j in range(_FW // 16):
                sv = src_st[p, wm, pl.ds(j * 16, 16)]
                dv = dst_st[p, wm, pl.ds(j * 16, 16)]
                m = (dv >= base) & (dv < base + _FS)
                rel = jnp.where(m, dv - base, _FS + j * 16 + lax.iota(jnp.int32, 16))
                srcw[b, pl.ds(j * 16, 16)] = sv + srcoff
                dstw[b, pl.ds(j * 16, 16)] = rel

        stage(0)
        fix(0, 0)
        pltpu.async_copy(gtab2.at[srcw.at[0]], rows.at[0], gsem.at[0])
        fix(1, 1)
        pltpu.async_copy(gtab2.at[srcw.at[1]], rows.at[1], gsem.at[1])

        def body(w, _):
            b = w % 2
            pltpu.make_async_copy(gtab2.at[srcw.at[b]], rows.at[b], gsem.at[b]).wait()
            pltpu.async_copy(rows.at[b], acc.at[dstw.at[b]], ssem.at[b], add=True).wait()

            @pl.when(w + 2 < _FWPT)
            def _():
                nw = w + 2

                @pl.when(nw % _FBLK == 0)
                def _():
                    stage(nw // _FBLK)

                fix(nw, b)
                pltpu.async_copy(gtab2.at[srcw.at[b]], rows.at[b], gsem.at[b])
            return ()

        lax.fori_loop(0, _FWPT, body, (), unroll=False)
        plsc.subcore_barrier()
        drpt = _FS // _NSUB
        pltpu.sync_copy(acc.at[pl.ds(s * drpt, drpt)],
                        out.at[c, d, pl.ds(s * drpt, drpt)])
        plsc.subcore_barrier()
        return ()

    lax.fori_loop(0, 2, half_pass, (), unroll=False)


def _f_agg(gtab, src_g, dst_g, init_full):
    src3d = src_g.reshape(_NSUB, _FWPT, _FW)
    dst3d = dst_g.reshape(_NSUB, _FWPT, _FW)
    gtab2 = jnp.concatenate([gtab[:, :_HF], gtab[:, _HF:]])
    zp = jnp.pad(init_full, ((0, _NFP - NF), (0, 0)))
    initp = jnp.zeros((2, 2, _FACC, _HF), jnp.float32).at[:, :, :_FS].set(
        jnp.stack([zp[:, :_HF], zp[:, _HF:]]).reshape(2, 2, _FS, _HF))
    k = pl.kernel(
        _f_agg_kernel,
        out_type=jax.ShapeDtypeStruct((2, 2, _FS, _HF), jnp.float32),
        mesh=plsc.VectorSubcoreMesh(core_axis_name="c", subcore_axis_name="s"),
        scratch_types=[
            pltpu.VMEM((2, _FBLK, _FW), jnp.int32),
            pltpu.VMEM((2, _FBLK, _FW), jnp.int32),
            pltpu.VMEM((2, _FW), jnp.int32),
            pltpu.VMEM((2, _FW), jnp.int32),
            pltpu.VMEM((2, _FW, _HF), jnp.float32),
            pltpu.SemaphoreType.DMA((2,)),
            pltpu.SemaphoreType.DMA((2,)),
            pltpu.VMEM_SHARED((_FACC, _HF), jnp.float32),
        ],
        compiler_params=pltpu.CompilerParams(use_tc_tiling_on_sc=False),
    )
    out = k(src3d, dst3d, initp, gtab2)
    return jnp.concatenate(
        [out[0].reshape(_NFP, _HF), out[1].reshape(_NFP, _HF)], axis=1)[:NF]


def _f2h_agg(msrc, e_src, e_dst, init_row):
    src_p, dst_p = _pad_edges(e_src, e_dst, NH, NF, _EPAD)
    src2d = src_p.reshape(_NW, _HWIN_PW, _W)
    dst2d = dst_p.reshape(_NW, _HWIN_PW, _W)
    init = jnp.stack([
        jnp.pad(init_row, ((0, _NHP - NH), (0, 0))),
        jnp.zeros((_NHP, H), jnp.float32),
    ])
    k = pl.kernel(
        _f2h_agg_kernel,
        out_type=jax.ShapeDtypeStruct((2, _NHP, H), jnp.float32),
        mesh=plsc.VectorSubcoreMesh(core_axis_name="c", subcore_axis_name="s"),
        scratch_types=[
            pltpu.VMEM((_HWIN_PW, _W), jnp.int32),
            pltpu.VMEM((_HWIN_PW, _W), jnp.int32),
            pltpu.VMEM((2, _W, H), jnp.float32),
            pltpu.SemaphoreType.DMA((2,)),
            pltpu.SemaphoreType.DMA((2,)),
            pltpu.VMEM_SHARED((_NHP, H), jnp.float32),
        ],
    )
    out = k(src2d, dst2d, init, msrc)
    return out[:, :NH]


def _gconv(x_src, x_dst, src, dst, Wrel, Wroot, b, n_dst):
    agg = jax.ops.segment_sum(x_src[src], dst, num_segments=n_dst)
    return agg @ Wrel + b + x_dst @ Wroot


def _mm_kernel(x_ref, w_ref, b_ref, o_ref, *, relu_x):
    x = x_ref[...]
    if relu_x:
        x = jnp.maximum(x, 0.0)
    o_ref[...] = jnp.dot(x, w_ref[...], preferred_element_type=jnp.float32) + b_ref[...]


def _mm(x, w, b, relu_x=False, bm=2000):
    m, kdim = x.shape
    n = w.shape[1]
    return pl.pallas_call(
        functools.partial(_mm_kernel, relu_x=relu_x),
        grid=(m // bm,),
        in_specs=[pl.BlockSpec((bm, kdim), lambda i: (i, 0)),
                  pl.BlockSpec((kdim, n), lambda i: (0, 0)),
                  pl.BlockSpec((1, n), lambda i: (0, 0))],
        out_specs=pl.BlockSpec((bm, n), lambda i: (i, 0)),
        out_shape=jax.ShapeDtypeStruct((m, n), jnp.float32),
    )(x, w, b.reshape(1, n))


def _h1_kernel(oh_ref, w_ref, o_ref):
    xh = jnp.maximum(oh_ref[0] + oh_ref[1], 0.0)
    o_ref[...] = jnp.dot(xh, w_ref[...], preferred_element_type=jnp.float32)


def _h1_mm(oh2, w):
    return pl.pallas_call(
        _h1_kernel,
        out_shape=jax.ShapeDtypeStruct((NH, H), jnp.float32),
    )(oh2, w)


_NFSEG = 51200


def _final_kernel(x_ref, bid_ref, w1_ref, b1_ref, w2_ref, b2_ref, w3_ref, b3_ref,
                  out_ref, acc_ref):
    i = pl.program_id(0)
    nblk = pl.num_programs(0)

    @pl.when(i == 0)
    def _():
        acc_ref[...] = jnp.full((NB, H), -jnp.inf, jnp.float32)

    x = x_ref[...]
    bid = bid_ref[...]
    lo = jnp.min(bid)
    hi = jnp.max(bid)
    for s in range(NB):
        @pl.when(jnp.logical_and(lo <= s, s <= hi))
        def _():
            m = jnp.max(jnp.where(bid == s, x, -jnp.inf), axis=0)
            acc_ref[s, :] = jnp.maximum(acc_ref[s, :], m)

    @pl.when(i == nblk - 1)
    def _():
        h1 = jnp.maximum(jnp.dot(acc_ref[...], w1_ref[...],
                                 preferred_element_type=jnp.float32) + b1_ref[...], 0.0)
        h2 = jnp.maximum(jnp.dot(h1, w2_ref[...],
                                 preferred_element_type=jnp.float32) + b2_ref[...], 0.0)
        out_ref[...] = jnp.dot(h2, w3_ref[...],
                               preferred_element_type=jnp.float32) + b3_ref[...]


def _segmax_mlp(x_f, nf_batch, w1, b1, w2, b2, w3, b3):
    xp = jnp.pad(x_f, ((0, _NFSEG - NF), (0, 0)))
    bidb = jnp.broadcast_to(
        jnp.pad(nf_batch.astype(jnp.float32), (0, _NFSEG - NF),
                constant_values=float(NB))[:, None], (_NFSEG, H))
    bm = 2048
    return pl.pallas_call(
        _final_kernel,
        grid=(_NFSEG // bm,),
        in_specs=[pl.BlockSpec((bm, H), lambda i: (i, 0)),
                  pl.BlockSpec((bm, H), lambda i: (i, 0)),
                  pl.BlockSpec((H, H // 2), lambda i: (0, 0)),
                  pl.BlockSpec((1, H // 2), lambda i: (0, 0)),
                  pl.BlockSpec((H // 2, H), lambda i: (0, 0)),
                  pl.BlockSpec((1, H), lambda i: (0, 0)),
                  pl.BlockSpec((H, NC), lambda i: (0, 0)),
                  pl.BlockSpec((1, NC), lambda i: (0, 0))],
        out_specs=pl.BlockSpec((NB, NC), lambda i: (0, 0)),
        out_shape=jax.ShapeDtypeStruct((NB, NC), jnp.float32),
        scratch_shapes=[pltpu.VMEM((NB, H), jnp.float32)],
    )(xp, bidb, w1, b1.reshape(1, -1), w2, b2.reshape(1, -1), w3, b3.reshape(1, -1))


def _hgather_kernel(ew, ids2d, out, idx_v, rows_v, sem):
    c = lax.axis_index("c")
    s = lax.axis_index("s")
    wid = c * _NSUB + s
    pltpu.sync_copy(ids2d.at[wid], idx_v)
    pltpu.async_copy(ew.at[idx_v], rows_v, sem).wait()
    pltpu.sync_copy(rows_v, out.at[wid])


def _hgather(ew, ids):
    idp = jnp.pad(ids.astype(jnp.int32), (0, 4096 - NH)).reshape(_NW, 128)
    k = pl.kernel(
        _hgather_kernel,
        out_type=jax.ShapeDtypeStruct((_NW, 128, 2 * H), jnp.float32),
        mesh=plsc.VectorSubcoreMesh(core_axis_name="c", subcore_axis_name="s"),
        scratch_types=[
            pltpu.VMEM((128,), jnp.int32),
            pltpu.VMEM((128, 2 * H), jnp.float32),
            pltpu.SemaphoreType.DMA,
        ],
    )
    return k(ew, idp).reshape(4096, 2 * H)


def kernel(host_node_ids, nf_x, e_h2f_src, e_h2f_dst, e_f2h_src, e_f2h_dst, e_f2f, nf_batch, emb_table, l0_h2f_Wrel, l0_h2f_Wroot, l0_h2f_b, l0_f2h_Wrel, l0_f2h_Wroot, l0_f2h_b, l0_f2f_Wrel, l0_f2f_Wroot, l0_f2f_b, l1_h2f_Wrel, l1_h2f_Wroot, l1_h2f_b, l1_f2h_Wrel, l1_f2h_Wroot, l1_f2h_b, l1_f2f_Wrel, l1_f2f_Wroot, l1_f2f_b, cls_W1, cls_b1, cls_W2, cls_b2, cls_W3, cls_b3):
    src_g, dst_g = _pad_edges(
        jnp.concatenate([e_h2f_src.astype(jnp.int32),
                         e_f2f[0].astype(jnp.int32) + NH]),
        jnp.concatenate([e_h2f_dst.astype(jnp.int32),
                         e_f2f[1].astype(jnp.int32)]),
        NF, NH + NF, _FEPAD)

    ew = _mm(emb_table, jnp.concatenate([l0_h2f_Wrel, l0_f2h_Wroot], axis=1),
             jnp.concatenate([jnp.zeros_like(l0_h2f_b), l0_f2h_b]), bm=4000)
    g = _hgather(ew, host_node_ids)
    f0 = _mm(nf_x,
             jnp.concatenate([l0_f2f_Wrel, l0_f2h_Wrel,
                              l0_h2f_Wroot + l0_f2f_Wroot], axis=1),
             jnp.concatenate([jnp.zeros((2 * H,), jnp.float32),
                              l0_h2f_b + l0_f2f_b]))

    gtab0 = jnp.concatenate([g[:NH, :H], f0[:, :H]])
    o_f0 = _f_agg(gtab0, src_g, dst_g, f0[:, 2 * H:])
    oh2 = _f2h_agg(f0[:, H:2 * H], e_f2h_src, e_f2h_dst, g[:NH, H:])

    t1 = _h1_mm(oh2, l1_h2f_Wrel)
    f1 = _mm(o_f0,
             jnp.concatenate([l1_f2f_Wrel, l1_h2f_Wroot + l1_f2f_Wroot], axis=1),
             jnp.concatenate([jnp.zeros((H,), jnp.float32),
                              l1_h2f_b + l1_f2f_b]), relu_x=True)
    gtab1 = jnp.concatenate([t1, f1[:, :H]])
    o_f1 = _f_agg(gtab1, src_g, dst_g, f1[:, H:])

    return _segmax_mlp(o_f1, nf_batch, cls_W1, cls_b1, cls_W2, cls_b2,
                       cls_W3, cls_b3)


# speedup vs baseline: 2.8656x; 1.0972x over previous
import functools

import jax
import jax.numpy as jnp
from jax import lax
from jax.experimental import pallas as pl
from jax.experimental.pallas import tpu as pltpu
from jax.experimental.pallas import tpu_sc as plsc

H = 128
F = 97
NH = 4000
NF = 50000
E = 500000
NC = 10
NB = 64

_NSC = 2
_NSUB = 16
_NW = _NSC * _NSUB

_W = 128
_HWIN_PW = -(-E // (_W * _NW))
_HWIN_TOT = _HWIN_PW * _NW
_EPAD = _HWIN_TOT * _W
_NHP = 4096


def _pad_edges(src, dst, n_dst_real, n_src_real, n_pad_total):
    npad = n_pad_total - src.shape[0]
    pad_pos = jnp.arange(npad, dtype=jnp.int32)
    pad_src = pad_pos * 37 % n_src_real
    pad_dst = n_dst_real + (pad_pos % 16)
    return (jnp.concatenate([src.astype(jnp.int32), pad_src]),
            jnp.concatenate([dst.astype(jnp.int32), pad_dst]))


def _f2h_agg_kernel(src2d, dst2d, init, msrc, out, src_st, dst_st, rows, gsem, ssem, acc):
    c = lax.axis_index("c")
    s = lax.axis_index("s")
    wid = c * _NSUB + s
    rows_per_tile = _NHP // _NSUB
    r0 = s * rows_per_tile

    pltpu.sync_copy(src2d.at[wid], src_st)
    pltpu.sync_copy(dst2d.at[wid], dst_st)
    pltpu.sync_copy(init.at[c, pl.ds(r0, rows_per_tile)], acc.at[pl.ds(r0, rows_per_tile)])
    plsc.subcore_barrier()

    pltpu.async_copy(msrc.at[src_st.at[0]], rows.at[0], gsem.at[0])
    pltpu.async_copy(msrc.at[src_st.at[1]], rows.at[1], gsem.at[1])

    def body(k, _):
        b = k % 2
        pltpu.make_async_copy(msrc.at[src_st.at[k]], rows.at[b], gsem.at[b]).wait()
        pltpu.async_copy(rows.at[b], acc.at[dst_st.at[k]], ssem.at[b], add=True).wait()

        @pl.when(k + 2 < _HWIN_PW)
        def _():
            pltpu.async_copy(msrc.at[src_st.at[k + 2]], rows.at[b], gsem.at[b])
        return ()

    lax.fori_loop(0, _HWIN_PW, body, (), unroll=False)
    plsc.subcore_barrier()
    pltpu.sync_copy(acc.at[pl.ds(r0, rows_per_tile)], out.at[c, pl.ds(r0, rows_per_tile)])


_FS = 25088
_FACC = _FS + 128
_NFP = 2 * _FS
_FW = 128
_FWPT = 512
_FBLK = 8
_FEPAD = _FWPT * _NSUB * _FW
_HF = H // 2


def _f_agg_kernel(src3d, dst3d, init, gtab2, out, src_st, dst_st, srcw, dstw,
                  rows, gsem, ssem, acc):
    c = lax.axis_index("c")
    s = lax.axis_index("s")
    rpt = _FACC // _NSUB
    r0 = s * rpt
    srcoff = c * (NH + NF)

    def half_pass(d, _):
        base = d * _FS
        pltpu.sync_copy(init.at[c, d, pl.ds(r0, rpt)], acc.at[pl.ds(r0, rpt)])
        plsc.subcore_barrier()

        def stage(blk):
            pltpu.sync_copy(src3d.at[s, pl.ds(blk * _FBLK, _FBLK)], src_st.at[blk % 2])
            pltpu.sync_copy(dst3d.at[s, pl.ds(blk * _FBLK, _FBLK)], dst_st.at[blk % 2])

        def fix(w, b):
            p = (w // _FBLK) % 2
            wm = w % _FBLK
            for j in range(_FW // 16):
                sv = src_st[p, wm, pl.ds(j * 16, 16)]
                dv = dst_st[p, wm, pl.ds(j * 16, 16)]
                m = (dv >= base) & (dv < base + _FS)
                rel = jnp.where(m, dv - base, _FS + j * 16 + lax.iota(jnp.int32, 16))
                srcw[b, pl.ds(j * 16, 16)] = sv + srcoff
                dstw[b, pl.ds(j * 16, 16)] = rel

        stage(0)
        fix(0, 0)
        pltpu.async_copy(gtab2.at[srcw.at[0]], rows.at[0], gsem.at[0])
        fix(1, 1)
        pltpu.async_copy(gtab2.at[srcw.at[1]], rows.at[1], gsem.at[1])

        def body(w, _):
            b = w % 2
            pltpu.make_async_copy(gtab2.at[srcw.at[b]], rows.at[b], gsem.at[b]).wait()
            pltpu.async_copy(rows.at[b], acc.at[dstw.at[b]], ssem.at[b], add=True).wait()

            @pl.when(w + 2 < _FWPT)
            def _():
                nw = w + 2

                @pl.when(nw % _FBLK == 0)
                def _():
                    stage(nw // _FBLK)

                fix(nw, b)
                pltpu.async_copy(gtab2.at[srcw.at[b]], rows.at[b], gsem.at[b])
            return ()

        lax.fori_loop(0, _FWPT, body, (), unroll=False)
        plsc.subcore_barrier()
        drpt = _FS // _NSUB
        pltpu.sync_copy(acc.at[pl.ds(s * drpt, drpt)],
                        out.at[c, d, pl.ds(s * drpt, drpt)])
        plsc.subcore_barrier()
        return ()

    lax.fori_loop(0, 2, half_pass, (), unroll=False)


def _f_agg(gtab, src_g, dst_g, init_full):
    src3d = src_g.reshape(_NSUB, _FWPT, _FW)
    dst3d = dst_g.reshape(_NSUB, _FWPT, _FW)
    gtab2 = jnp.concatenate([gtab[:, :_HF], gtab[:, _HF:]])
    zp = jnp.pad(init_full, ((0, _NFP - NF), (0, 0)))
    initp = jnp.zeros((2, 2, _FACC, _HF), jnp.float32).at[:, :, :_FS].set(
        jnp.stack([zp[:, :_HF], zp[:, _HF:]]).reshape(2, 2, _FS, _HF))
    k = pl.kernel(
        _f_agg_kernel,
        out_type=jax.ShapeDtypeStruct((2, 2, _FS, _HF), jnp.float32),
        mesh=plsc.VectorSubcoreMesh(core_axis_name="c", subcore_axis_name="s"),
        scratch_types=[
            pltpu.VMEM((2, _FBLK, _FW), jnp.int32),
            pltpu.VMEM((2, _FBLK, _FW), jnp.int32),
            pltpu.VMEM((2, _FW), jnp.int32),
            pltpu.VMEM((2, _FW), jnp.int32),
            pltpu.VMEM((2, _FW, _HF), jnp.float32),
            pltpu.SemaphoreType.DMA((2,)),
            pltpu.SemaphoreType.DMA((2,)),
            pltpu.VMEM_SHARED((_FACC, _HF), jnp.float32),
        ],
        compiler_params=pltpu.CompilerParams(use_tc_tiling_on_sc=False),
    )
    out = k(src3d, dst3d, initp, gtab2)
    return jnp.concatenate(
        [out[0].reshape(_NFP, _HF), out[1].reshape(_NFP, _HF)], axis=1)[:NF]


def _f2h_agg(msrc, e_src, e_dst, init_row):
    src_p, dst_p = _pad_edges(e_src, e_dst, NH, NF, _EPAD)
    src2d = src_p.reshape(_NW, _HWIN_PW, _W)
    dst2d = dst_p.reshape(_NW, _HWIN_PW, _W)
    init = jnp.stack([
        jnp.pad(init_row, ((0, _NHP - NH), (0, 0))),
        jnp.zeros((_NHP, H), jnp.float32),
    ])
    k = pl.kernel(
        _f2h_agg_kernel,
        out_type=jax.ShapeDtypeStruct((2, _NHP, H), jnp.float32),
        mesh=plsc.VectorSubcoreMesh(core_axis_name="c", subcore_axis_name="s"),
        scratch_types=[
            pltpu.VMEM((_HWIN_PW, _W), jnp.int32),
            pltpu.VMEM((_HWIN_PW, _W), jnp.int32),
            pltpu.VMEM((2, _W, H), jnp.float32),
            pltpu.SemaphoreType.DMA((2,)),
            pltpu.SemaphoreType.DMA((2,)),
            pltpu.VMEM_SHARED((_NHP, H), jnp.float32),
        ],
    )
    out = k(src2d, dst2d, init, msrc)
    return out[:, :NH]


def _gconv(x_src, x_dst, src, dst, Wrel, Wroot, b, n_dst):
    agg = jax.ops.segment_sum(x_src[src], dst, num_segments=n_dst)
    return agg @ Wrel + b + x_dst @ Wroot


def _mm_kernel(x_ref, w_ref, b_ref, o_ref, *, relu_x):
    x = x_ref[...]
    if relu_x:
        x = jnp.maximum(x, 0.0)
    o_ref[...] = jnp.dot(x, w_ref[...], preferred_element_type=jnp.float32) + b_ref[...]


def _mm(x, w, b, relu_x=False, bm=2000):
    m, kdim = x.shape
    n = w.shape[1]
    return pl.pallas_call(
        functools.partial(_mm_kernel, relu_x=relu_x),
        grid=(m // bm,),
        in_specs=[pl.BlockSpec((bm, kdim), lambda i: (i, 0)),
                  pl.BlockSpec((kdim, n), lambda i: (0, 0)),
                  pl.BlockSpec((1, n), lambda i: (0, 0))],
        out_specs=pl.BlockSpec((bm, n), lambda i: (i, 0)),
        out_shape=jax.ShapeDtypeStruct((m, n), jnp.float32),
    )(x, w, b.reshape(1, n))


def _h1_kernel(oh_ref, w_ref, o_ref):
    xh = jnp.maximum(oh_ref[0] + oh_ref[1], 0.0)
    o_ref[...] = jnp.dot(xh, w_ref[...], preferred_element_type=jnp.float32)


def _h1_mm(oh2, w):
    return pl.pallas_call(
        _h1_kernel,
        out_shape=jax.ShapeDtypeStruct((NH, H), jnp.float32),
    )(oh2, w)


_NFSEG = 51200


def _final_kernel(x_ref, bid_ref, w1_ref, b1_ref, w2_ref, b2_ref, w3_ref, b3_ref,
                  out_ref, acc_ref):
    i = pl.program_id(0)
    nblk = pl.num_programs(0)

    @pl.when(i == 0)
    def _():
        acc_ref[...] = jnp.full((NB, H), -jnp.inf, jnp.float32)

    x = x_ref[...]
    bid = bid_ref[...]
    lo = jnp.min(bid)
    hi = jnp.max(bid)
    for s in range(NB):
        @pl.when(jnp.logical_and(lo <= s, s <= hi))
        def _():
            m = jnp.max(jnp.where(bid == s, x, -jnp.inf), axis=0)
            acc_ref[s, :] = jnp.maximum(acc_ref[s, :], m)

    @pl.when(i == nblk - 1)
    def _():
        h1 = jnp.maximum(jnp.dot(acc_ref[...], w1_ref[...],
                                 preferred_element_type=jnp.float32) + b1_ref[...], 0.0)
        h2 = jnp.maximum(jnp.dot(h1, w2_ref[...],
                                 preferred_element_type=jnp.float32) + b2_ref[...], 0.0)
        out_ref[...] = jnp.dot(h2, w3_ref[...],
                               preferred_element_type=jnp.float32) + b3_ref[...]


def _segmax_mlp(x_f, nf_batch, w1, b1, w2, b2, w3, b3):
    xp = jnp.pad(x_f, ((0, _NFSEG - NF), (0, 0)))
    bidb = jnp.broadcast_to(
        jnp.pad(nf_batch.astype(jnp.float32), (0, _NFSEG - NF),
                constant_values=float(NB))[:, None], (_NFSEG, H))
    bm = 2048
    return pl.pallas_call(
        _final_kernel,
        grid=(_NFSEG // bm,),
        in_specs=[pl.BlockSpec((bm, H), lambda i: (i, 0)),
                  pl.BlockSpec((bm, H), lambda i: (i, 0)),
                  pl.BlockSpec((H, H // 2), lambda i: (0, 0)),
                  pl.BlockSpec((1, H // 2), lambda i: (0, 0)),
                  pl.BlockSpec((H // 2, H), lambda i: (0, 0)),
                  pl.BlockSpec((1, H), lambda i: (0, 0)),
                  pl.BlockSpec((H, NC), lambda i: (0, 0)),
                  pl.BlockSpec((1, NC), lambda i: (0, 0))],
        out_specs=pl.BlockSpec((NB, NC), lambda i: (0, 0)),
        out_shape=jax.ShapeDtypeStruct((NB, NC), jnp.float32),
        scratch_shapes=[pltpu.VMEM((NB, H), jnp.float32)],
    )(xp, bidb, w1, b1.reshape(1, -1), w2, b2.reshape(1, -1), w3, b3.reshape(1, -1))


def _hgather_kernel(ew, ids2d, out, idx_v, rows_v, sem):
    c = lax.axis_index("c")
    s = lax.axis_index("s")
    wid = c * _NSUB + s
    pltpu.sync_copy(ids2d.at[wid], idx_v)
    pltpu.async_copy(ew.at[idx_v], rows_v, sem).wait()
    pltpu.sync_copy(rows_v, out.at[wid])


def _hgather(ew, ids):
    idp = jnp.pad(ids.astype(jnp.int32), (0, 4096 - NH)).reshape(_NW, 128)
    k = pl.kernel(
        _hgather_kernel,
        out_type=jax.ShapeDtypeStruct((_NW, 128, 2 * H), jnp.float32),
        mesh=plsc.VectorSubcoreMesh(core_axis_name="c", subcore_axis_name="s"),
        scratch_types=[
            pltpu.VMEM((128,), jnp.int32),
            pltpu.VMEM((128, 2 * H), jnp.float32),
            pltpu.SemaphoreType.DMA,
        ],
    )
    return k(ew, idp).reshape(4096, 2 * H)


def kernel(host_node_ids, nf_x, e_h2f_src, e_h2f_dst, e_f2h_src, e_f2h_dst, e_f2f, nf_batch, emb_table, l0_h2f_Wrel, l0_h2f_Wroot, l0_h2f_b, l0_f2h_Wrel, l0_f2h_Wroot, l0_f2h_b, l0_f2f_Wrel, l0_f2f_Wroot, l0_f2f_b, l1_h2f_Wrel, l1_h2f_Wroot, l1_h2f_b, l1_f2h_Wrel, l1_f2h_Wroot, l1_f2h_b, l1_f2f_Wrel, l1_f2f_Wroot, l1_f2f_b, cls_W1, cls_b1, cls_W2, cls_b2, cls_W3, cls_b3):
    src_g, dst_g = _pad_edges(
        jnp.concatenate([e_h2f_src.astype(jnp.int32),
                         e_f2f[0].astype(jnp.int32) + NH]),
        jnp.concatenate([e_h2f_dst.astype(jnp.int32),
                         e_f2f[1].astype(jnp.int32)]),
        NF, NH + NF, _FEPAD)

    ew = _mm(emb_table, jnp.concatenate([l0_h2f_Wrel, l0_f2h_Wroot], axis=1),
             jnp.concatenate([jnp.zeros_like(l0_h2f_b), l0_f2h_b]), bm=4000)
    g = _hgather(ew, host_node_ids)
    f0 = _mm(nf_x,
             jnp.concatenate([l0_f2f_Wrel, l0_f2h_Wrel,
                              l0_h2f_Wroot + l0_f2f_Wroot], axis=1),
             jnp.concatenate([jnp.zeros((2 * H,), jnp.float32),
                              l0_h2f_b + l0_f2f_b]))

    gtab0 = jnp.concatenate([g[:NH, :H], f0[:, :H]])
    o_f0 = _f_agg(gtab0, src_g, dst_g, f0[:, 2 * H:])
    oh2 = _f2h_agg(f0[:, H:2 * H], e_f2h_src, e_f2h_dst, g[:NH, H:])

    t1 = _h1_mm(oh2, l1_h2f_Wrel)
    f1 = _mm(o_f0,
             jnp.concatenate([l1_f2f_Wrel, l1_h2f_Wroot + l1_f2f_Wroot], axis=1),
             jnp.concatenate([jnp.zeros((H,), jnp.float32),
                              l1_h2f_b + l1_f2f_b]), relu_x=True)
    gtab1 = jnp.concatenate([t1, f1[:, :H]])
    o_f1 = _f_agg(gtab1, src_g, dst_g, f1[:, H:])

    return _segmax_mlp(o_f1, nf_batch, cls_W1, cls_b1, cls_W2, cls_b2,
                       cls_W3, cls_b3)


# speedup vs baseline: 2.8680x; 1.0008x over previous
import functools

import jax
import jax.numpy as jnp
from jax import lax
from jax.experimental import pallas as pl
from jax.experimental.pallas import tpu as pltpu
from jax.experimental.pallas import tpu_sc as plsc

H = 128
F = 97
NH = 4000
NF = 50000
E = 500000
NC = 10
NB = 64

_NSC = 2
_NSUB = 16
_NW = _NSC * _NSUB

_W = 128
_HWIN_PW = -(-E // (_W * _NW))
_HWIN_TOT = _HWIN_PW * _NW
_EPAD = _HWIN_TOT * _W
_NHP = 4096


def _pad_edges(src, dst, n_dst_real, n_src_real, n_pad_total):
    npad = n_pad_total - src.shape[0]
    pad_pos = jnp.arange(npad, dtype=jnp.int32)
    pad_src = pad_pos * 37 % n_src_real
    pad_dst = n_dst_real + (pad_pos % 16)
    return (jnp.concatenate([src.astype(jnp.int32), pad_src]),
            jnp.concatenate([dst.astype(jnp.int32), pad_dst]))


def _f2h_agg_kernel(src2d, dst2d, init, msrc, out, src_st, dst_st, rows, gsem, ssem, acc):
    c = lax.axis_index("c")
    s = lax.axis_index("s")
    wid = c * _NSUB + s
    rows_per_tile = _NHP // _NSUB
    r0 = s * rows_per_tile

    pltpu.sync_copy(src2d.at[wid], src_st)
    pltpu.sync_copy(dst2d.at[wid], dst_st)
    pltpu.sync_copy(init.at[c, pl.ds(r0, rows_per_tile)], acc.at[pl.ds(r0, rows_per_tile)])
    plsc.subcore_barrier()

    pltpu.async_copy(msrc.at[src_st.at[0]], rows.at[0], gsem.at[0])
    pltpu.async_copy(msrc.at[src_st.at[1]], rows.at[1], gsem.at[1])

    def body(k, _):
        b = k % 2
        pltpu.make_async_copy(msrc.at[src_st.at[k]], rows.at[b], gsem.at[b]).wait()
        pltpu.async_copy(rows.at[b], acc.at[dst_st.at[k]], ssem.at[b], add=True).wait()

        @pl.when(k + 2 < _HWIN_PW)
        def _():
            pltpu.async_copy(msrc.at[src_st.at[k + 2]], rows.at[b], gsem.at[b])
        return ()

    lax.fori_loop(0, _HWIN_PW, body, (), unroll=False)
    plsc.subcore_barrier()
    pltpu.sync_copy(acc.at[pl.ds(r0, rows_per_tile)], out.at[c, pl.ds(r0, rows_per_tile)])


_FS = 25088
_FACC = _FS + 128
_NFP = 2 * _FS
_FW = 128
_FWPT = 512
_FBLK = 8
_FEPAD = _FWPT * _NSUB * _FW
_HF = H // 2


def _f_agg_kernel(src3d, dst3d, init, gtab2, out, src_st, dst_st, srcw, dstw,
                  rows, gsem, ssem, acc):
    c = lax.axis_index("c")
    s = lax.axis_index("s")
    rpt = _FACC // _NSUB
    r0 = s * rpt
    srcoff = c * (NH + NF)

    def half_pass(d, _):
        base = d * _FS
        pltpu.sync_copy(init.at[c, d, pl.ds(r0, rpt)], acc.at[pl.ds(r0, rpt)])
        plsc.subcore_barrier()

        def stage(blk):
            pltpu.sync_copy(src3d.at[s, pl.ds(blk * _FBLK, _FBLK)], src_st.at[blk % 2])
            pltpu.sync_copy(dst3d.at[s, pl.ds(blk * _FBLK, _FBLK)], dst_st.at[blk % 2])

        def fix(w, b):
            p = (w // _FBLK) % 2
            wm = w % _FBLK
            for j in range(_FW // 16):
                sv = src_st[p, wm, pl.ds(j * 16, 16)]
                dv = dst_st[p, wm, pl.ds(j * 16, 16)]
                m = (dv >= base) & (dv < base + _FS)
                rel = jnp.where(m, dv - base, _FS + j * 16 + lax.iota(jnp.int32, 16))
                srcw[b, pl.ds(j * 16, 16)] = sv + srcoff
                dstw[b, pl.ds(j * 16, 16)] = rel

        stage(0)
        fix(0, 0)
        pltpu.async_copy(gtab2.at[srcw.at[0]], rows.at[0], gsem.at[0])
        fix(1, 1)
        pltpu.async_copy(gtab2.at[srcw.at[1]], rows.at[1], gsem.at[1])

        def body(w, _):
            b = w % 2
            pltpu.make_async_copy(gtab2.at[srcw.at[b]], rows.at[b], gsem.at[b]).wait()
            pltpu.async_copy(rows.at[b], acc.at[dstw.at[b]], ssem.at[b], add=True).wait()

            @pl.when(w + 2 < _FWPT)
            def _():
                nw = w + 2

                @pl.when(nw % _FBLK == 0)
                def _():
                    stage(nw // _FBLK)

                fix(nw, b)
                pltpu.async_copy(gtab2.at[srcw.at[b]], rows.at[b], gsem.at[b])
            return ()

        lax.fori_loop(0, _FWPT, body, (), unroll=False)
        plsc.subcore_barrier()
        drpt = _FS // _NSUB
        pltpu.sync_copy(acc.at[pl.ds(s * drpt, drpt)],
                        out.at[c, d, pl.ds(s * drpt, drpt)])
        plsc.subcore_barrier()
        return ()

    lax.fori_loop(0, 2, half_pass, (), unroll=False)


def _f_agg(gtab, src_g, dst_g, init_full):
    src3d = src_g.reshape(_NSUB, _FWPT, _FW)
    dst3d = dst_g.reshape(_NSUB, _FWPT, _FW)
    gtab2 = jnp.concatenate(gtab)
    zp = jnp.pad(init_full, ((0, _NFP - NF), (0, 0)))
    initp = jnp.concatenate(
        [jnp.stack([zp[:, :_HF], zp[:, _HF:]]).reshape(2, 2, _FS, _HF),
         jnp.zeros((2, 2, 128, _HF), jnp.float32)], axis=2)
    k = pl.kernel(
        _f_agg_kernel,
        out_type=jax.ShapeDtypeStruct((2, 2, _FS, _HF), jnp.float32),
        mesh=plsc.VectorSubcoreMesh(core_axis_name="c", subcore_axis_name="s"),
        scratch_types=[
            pltpu.VMEM((2, _FBLK, _FW), jnp.int32),
            pltpu.VMEM((2, _FBLK, _FW), jnp.int32),
            pltpu.VMEM((2, _FW), jnp.int32),
            pltpu.VMEM((2, _FW), jnp.int32),
            pltpu.VMEM((2, _FW, _HF), jnp.float32),
            pltpu.SemaphoreType.DMA((2,)),
            pltpu.SemaphoreType.DMA((2,)),
            pltpu.VMEM_SHARED((_FACC, _HF), jnp.float32),
        ],
        compiler_params=pltpu.CompilerParams(use_tc_tiling_on_sc=False),
    )
    out = k(src3d, dst3d, initp, gtab2)
    return jnp.concatenate(
        [out[0].reshape(_NFP, _HF), out[1].reshape(_NFP, _HF)], axis=1)[:NF]


def _f2h_agg(msrc, e_src, e_dst, init_row):
    src_p, dst_p = _pad_edges(e_src, e_dst, NH, NF, _EPAD)
    src2d = src_p.reshape(_NW, _HWIN_PW, _W)
    dst2d = dst_p.reshape(_NW, _HWIN_PW, _W)
    init = jnp.stack([
        jnp.pad(init_row, ((0, _NHP - NH), (0, 0))),
        jnp.zeros((_NHP, H), jnp.float32),
    ])
    k = pl.kernel(
        _f2h_agg_kernel,
        out_type=jax.ShapeDtypeStruct((2, _NHP, H), jnp.float32),
        mesh=plsc.VectorSubcoreMesh(core_axis_name="c", subcore_axis_name="s"),
        scratch_types=[
            pltpu.VMEM((_HWIN_PW, _W), jnp.int32),
            pltpu.VMEM((_HWIN_PW, _W), jnp.int32),
            pltpu.VMEM((2, _W, H), jnp.float32),
            pltpu.SemaphoreType.DMA((2,)),
            pltpu.SemaphoreType.DMA((2,)),
            pltpu.VMEM_SHARED((_NHP, H), jnp.float32),
        ],
    )
    out = k(src2d, dst2d, init, msrc)
    return out[:, :NH]


def _gconv(x_src, x_dst, src, dst, Wrel, Wroot, b, n_dst):
    agg = jax.ops.segment_sum(x_src[src], dst, num_segments=n_dst)
    return agg @ Wrel + b + x_dst @ Wroot


def _mm_kernel(x_ref, w_ref, b_ref, o_ref, *, relu_x):
    x = x_ref[...]
    if relu_x:
        x = jnp.maximum(x, 0.0)
    o_ref[...] = jnp.dot(x, w_ref[...], preferred_element_type=jnp.float32) + b_ref[...]


def _mm(x, w, b, relu_x=False, bm=2000):
    m, kdim = x.shape
    n = w.shape[1]
    return pl.pallas_call(
        functools.partial(_mm_kernel, relu_x=relu_x),
        grid=(m // bm,),
        in_specs=[pl.BlockSpec((bm, kdim), lambda i: (i, 0)),
                  pl.BlockSpec((kdim, n), lambda i: (0, 0)),
                  pl.BlockSpec((1, n), lambda i: (0, 0))],
        out_specs=pl.BlockSpec((bm, n), lambda i: (i, 0)),
        out_shape=jax.ShapeDtypeStruct((m, n), jnp.float32),
    )(x, w, b.reshape(1, n))


def _h1_kernel(oh_ref, w_ref, o_ref):
    xh = jnp.maximum(oh_ref[0] + oh_ref[1], 0.0)
    o_ref[...] = jnp.dot(xh, w_ref[...], preferred_element_type=jnp.float32)


def _h1_mm(oh2, w):
    return pl.pallas_call(
        _h1_kernel,
        out_shape=jax.ShapeDtypeStruct((NH, H), jnp.float32),
    )(oh2, w)


_NFSEG = 51200


def _final_kernel(x_ref, bid_ref, w1_ref, b1_ref, w2_ref, b2_ref, w3_ref, b3_ref,
                  out_ref, acc_ref):
    i = pl.program_id(0)
    nblk = pl.num_programs(0)

    @pl.when(i == 0)
    def _():
        acc_ref[...] = jnp.full((NB, H), -jnp.inf, jnp.float32)

    x = x_ref[...]
    bid = bid_ref[...]
    lo = jnp.min(bid)
    hi = jnp.max(bid)
    for s in range(NB):
        @pl.when(jnp.logical_and(lo <= s, s <= hi))
        def _():
            m = jnp.max(jnp.where(bid == s, x, -jnp.inf), axis=0)
            acc_ref[s, :] = jnp.maximum(acc_ref[s, :], m)

    @pl.when(i == nblk - 1)
    def _():
        h1 = jnp.maximum(jnp.dot(acc_ref[...], w1_ref[...],
                                 preferred_element_type=jnp.float32) + b1_ref[...], 0.0)
        h2 = jnp.maximum(jnp.dot(h1, w2_ref[...],
                                 preferred_element_type=jnp.float32) + b2_ref[...], 0.0)
        out_ref[...] = jnp.dot(h2, w3_ref[...],
                               preferred_element_type=jnp.float32) + b3_ref[...]


def _segmax_mlp(x_f, nf_batch, w1, b1, w2, b2, w3, b3):
    xp = jnp.pad(x_f, ((0, _NFSEG - NF), (0, 0)))
    bidb = jnp.broadcast_to(
        jnp.pad(nf_batch.astype(jnp.float32), (0, _NFSEG - NF),
                constant_values=float(NB))[:, None], (_NFSEG, H))
    bm = 2048
    return pl.pallas_call(
        _final_kernel,
        grid=(_NFSEG // bm,),
        in_specs=[pl.BlockSpec((bm, H), lambda i: (i, 0)),
                  pl.BlockSpec((bm, H), lambda i: (i, 0)),
                  pl.BlockSpec((H, H // 2), lambda i: (0, 0)),
                  pl.BlockSpec((1, H // 2), lambda i: (0, 0)),
                  pl.BlockSpec((H // 2, H), lambda i: (0, 0)),
                  pl.BlockSpec((1, H), lambda i: (0, 0)),
                  pl.BlockSpec((H, NC), lambda i: (0, 0)),
                  pl.BlockSpec((1, NC), lambda i: (0, 0))],
        out_specs=pl.BlockSpec((NB, NC), lambda i: (0, 0)),
        out_shape=jax.ShapeDtypeStruct((NB, NC), jnp.float32),
        scratch_shapes=[pltpu.VMEM((NB, H), jnp.float32)],
    )(xp, bidb, w1, b1.reshape(1, -1), w2, b2.reshape(1, -1), w3, b3.reshape(1, -1))


def _hgather_kernel(ew, ids2d, out, idx_v, rows_v, sem):
    c = lax.axis_index("c")
    s = lax.axis_index("s")
    wid = c * _NSUB + s
    pltpu.sync_copy(ids2d.at[wid], idx_v)
    pltpu.async_copy(ew.at[idx_v], rows_v, sem).wait()
    pltpu.sync_copy(rows_v, out.at[wid])


def _hgather(ew, ids):
    idp = jnp.pad(ids.astype(jnp.int32), (0, 4096 - NH)).reshape(_NW, 128)
    k = pl.kernel(
        _hgather_kernel,
        out_type=jax.ShapeDtypeStruct((_NW, 128, 2 * H), jnp.float32),
        mesh=plsc.VectorSubcoreMesh(core_axis_name="c", subcore_axis_name="s"),
        scratch_types=[
            pltpu.VMEM((128,), jnp.int32),
            pltpu.VMEM((128, 2 * H), jnp.float32),
            pltpu.SemaphoreType.DMA,
        ],
    )
    return k(ew, idp).reshape(4096, 2 * H)


def kernel(host_node_ids, nf_x, e_h2f_src, e_h2f_dst, e_f2h_src, e_f2h_dst, e_f2f, nf_batch, emb_table, l0_h2f_Wrel, l0_h2f_Wroot, l0_h2f_b, l0_f2h_Wrel, l0_f2h_Wroot, l0_f2h_b, l0_f2f_Wrel, l0_f2f_Wroot, l0_f2f_b, l1_h2f_Wrel, l1_h2f_Wroot, l1_h2f_b, l1_f2h_Wrel, l1_f2h_Wroot, l1_f2h_b, l1_f2f_Wrel, l1_f2f_Wroot, l1_f2f_b, cls_W1, cls_b1, cls_W2, cls_b2, cls_W3, cls_b3):
    src_g, dst_g = _pad_edges(
        jnp.concatenate([e_h2f_src.astype(jnp.int32),
                         e_f2f[0].astype(jnp.int32) + NH]),
        jnp.concatenate([e_h2f_dst.astype(jnp.int32),
                         e_f2f[1].astype(jnp.int32)]),
        NF, NH + NF, _FEPAD)

    ew = _mm(emb_table, jnp.concatenate([l0_h2f_Wrel, l0_f2h_Wroot], axis=1),
             jnp.concatenate([jnp.zeros_like(l0_h2f_b), l0_f2h_b]), bm=4000)
    g = _hgather(ew, host_node_ids)
    f0 = _mm(nf_x,
             jnp.concatenate([l0_f2f_Wrel, l0_f2h_Wrel,
                              l0_h2f_Wroot + l0_f2f_Wroot], axis=1),
             jnp.concatenate([jnp.zeros((2 * H,), jnp.float32),
                              l0_h2f_b + l0_f2f_b]))

    gtab0 = [g[:NH, :_HF], f0[:, :_HF], g[:NH, _HF:H], f0[:, _HF:H]]
    o_f0 = _f_agg(gtab0, src_g, dst_g, f0[:, 2 * H:])
    oh2 = _f2h_agg(f0[:, H:2 * H], e_f2h_src, e_f2h_dst, g[:NH, H:])

    t1 = _h1_mm(oh2, l1_h2f_Wrel)
    f1 = _mm(o_f0,
             jnp.concatenate([l1_f2f_Wrel, l1_h2f_Wroot + l1_f2f_Wroot], axis=1),
             jnp.concatenate([jnp.zeros((H,), jnp.float32),
                              l1_h2f_b + l1_f2f_b]), relu_x=True)
    gtab1 = [t1[:, :_HF], f1[:, :_HF], t1[:, _HF:], f1[:, _HF:H]]
    o_f1 = _f_agg(gtab1, src_g, dst_g, f1[:, H:])

    return _segmax_mlp(o_f1, nf_batch, cls_W1, cls_b1, cls_W2, cls_b2,
                       cls_W3, cls_b3)
